# baseline (speedup 1.0000x reference)
import jax
import jax.numpy as jnp
from jax.experimental import pallas as pl
from jax.experimental.pallas import tpu as pltpu

E = 8
TOPK = 2
D = 1024
I = 2048
IS = 4096
S = 2048
BT = 256
NB = S * TOPK // BT + E
NPAD = NB * BT
TB = S // BT

_F32 = jnp.float32
_BF16 = jnp.bfloat16


def _router_body(x_ref, wgt_ref, xbf_ref, p0_ref, p1_ref, s0_ref, s1_ref,
                 be_ref):
    x = x_ref[...]
    xbf_ref[...] = x.astype(_BF16)

    logits = jnp.dot(x, wgt_ref[...], preferred_element_type=_F32)
    iota_e = jax.lax.broadcasted_iota(jnp.int32, (S, E), 1)
    v0 = jnp.max(logits, axis=1, keepdims=True)
    i0 = jnp.min(jnp.where(logits == v0, iota_e, E), axis=1, keepdims=True)
    masked = jnp.where(iota_e == i0, -jnp.inf, logits)
    v1 = jnp.max(masked, axis=1, keepdims=True)
    i1 = jnp.min(jnp.where(masked == v1, iota_e, E), axis=1, keepdims=True)
    s0 = 1.0 / (1.0 + jnp.exp(v1 - v0))
    s0_ref[...] = s0
    s1_ref[...] = 1.0 - s0

    oh0 = (iota_e == i0).astype(_BF16)
    oh1 = (iota_e == i1).astype(_BF16)
    ohsum = oh0 + oh1
    iota_r = jax.lax.broadcasted_iota(jnp.int32, (S, S), 0)
    iota_c = jax.lax.broadcasted_iota(jnp.int32, (S, S), 1)
    tri = (iota_r > iota_c).astype(_BF16)
    cum_excl = jnp.dot(tri, ohsum, preferred_element_type=_F32)
    counts = jnp.sum(ohsum.astype(_F32), axis=0, keepdims=True)
    rank0 = jnp.sum(jnp.where(iota_e == i0, cum_excl, 0.0), axis=1,
                    keepdims=True)
    rank1 = jnp.sum(jnp.where(iota_e == i1, cum_excl, 0.0), axis=1,
                    keepdims=True)
    blocks = jnp.floor((counts + (BT - 1)) * (1.0 / BT))
    iota_ec = jax.lax.broadcasted_iota(jnp.int32, (E, E), 0)
    iota_er = jax.lax.broadcasted_iota(jnp.int32, (E, E), 1)
    tlow = (iota_ec <= iota_er).astype(_F32)
    ends_blocks = jnp.dot(blocks, tlow, preferred_element_type=_F32)
    base_rows = (ends_blocks - blocks) * BT
    base_b = jnp.broadcast_to(base_rows, (S, E))
    base0 = jnp.sum(jnp.where(iota_e == i0, base_b, 0.0), axis=1,
                    keepdims=True)
    base1 = jnp.sum(jnp.where(iota_e == i1, base_b, 0.0), axis=1,
                    keepdims=True)
    p0_ref[...] = (base0 + rank0).astype(jnp.int32)
    p1_ref[...] = (base1 + rank1).astype(jnp.int32)

    iota_g = jax.lax.broadcasted_iota(jnp.int32, (128, E), 0)
    ends_b = jnp.broadcast_to(ends_blocks, (128, E))
    be_ref[...] = jnp.sum(
        (iota_g.astype(_F32) >= ends_b).astype(jnp.int32),
        axis=1, keepdims=True)


def _router(xf, w_gate_t):
    return pl.pallas_call(
        _router_body,
        out_shape=(
            jax.ShapeDtypeStruct((S, D), _BF16),
            jax.ShapeDtypeStruct((S, 1), jnp.int32),
            jax.ShapeDtypeStruct((S, 1), jnp.int32),
            jax.ShapeDtypeStruct((S, 1), _F32),
            jax.ShapeDtypeStruct((S, 1), _F32),
            jax.ShapeDtypeStruct((128, 1), jnp.int32),
        ),
    )(xf, w_gate_t)


def _expert_body(be_ref, p0_ref, p1_ref, xf_ref, w1_ref, w2_ref, out_ref):
    g = pl.program_id(0)

    @pl.when(be_ref[g, 0] >= E)
    def _():
        out_ref[...] = jnp.zeros((BT, D), _BF16)

    @pl.when(be_ref[g, 0] < E)
    def _():
        rowpos = g * BT + jax.lax.broadcasted_iota(jnp.int32, (BT, 1), 0)
        p0 = p0_ref[...]
        p1 = p1_ref[...]
        onehot = ((p0 == rowpos).astype(_BF16)
                  + (p1 == rowpos).astype(_BF16))
        xg = jnp.dot(onehot, xf_ref[...], preferred_element_type=_F32)
        w1b = w1_ref[0].astype(_BF16)
        h = jnp.dot(xg.astype(_BF16), w1b, preferred_element_type=_F32)
        a = h[:, :I]
        b = h[:, I:]
        act = (a * jax.nn.sigmoid(a) * b).astype(_BF16)
        out_ref[...] = jnp.dot(act, w2_ref[0],
                               preferred_element_type=_F32).astype(_BF16)


def _expert_mlp(block_expert, p0l, p1l, xf_bf, w1_bf, w2_bf):
    grid_spec = pltpu.PrefetchScalarGridSpec(
        num_scalar_prefetch=1,
        grid=(NB,),
        in_specs=[
            pl.BlockSpec((1, S), lambda g, be: (0, 0)),
            pl.BlockSpec((1, S), lambda g, be: (0, 0)),
            pl.BlockSpec((S, D), lambda g, be: (0, 0)),
            pl.BlockSpec((1, D, 2 * I),
                         lambda g, be: (jnp.minimum(be[g, 0], E - 1), 0, 0)),
            pl.BlockSpec((1, I, D),
                         lambda g, be: (jnp.minimum(be[g, 0], E - 1), 0, 0)),
        ],
        out_specs=pl.BlockSpec((BT, D), lambda g, be: (g, 0)),
    )
    return pl.pallas_call(
        _expert_body,
        grid_spec=grid_spec,
        out_shape=jax.ShapeDtypeStruct((NPAD, D), _BF16),
        compiler_params=pltpu.CompilerParams(
            dimension_semantics=("parallel",),
        ),
    )(block_expert, p0l, p1l, xf_bf, w1_bf, w2_bf)


ISC = 512
JS = IS // ISC


def _shared_body(x_ref, wg_ref, wu_ref, wd_ref, eo_ref, p0_ref, p1_ref,
                 s0_ref, s1_ref, o_ref):
    j = pl.program_id(0)
    x = x_ref[...]
    wgj = wg_ref[...].astype(_BF16)
    wuj = wu_ref[...].astype(_BF16)
    wdj = wd_ref[...].astype(_BF16)
    g_ = jnp.dot(x, wgj, preferred_element_type=_F32)
    u_ = jnp.dot(x, wuj, preferred_element_type=_F32)
    act = (g_ * jax.lax.logistic(g_) * u_).astype(_BF16)
    partial = jnp.dot(act, wdj, preferred_element_type=_F32)

    @pl.when(j == 0)
    def _():
        o_ref[...] = partial

    @pl.when(j > 0)
    def _():
        o_ref[...] += partial

    @pl.when(j == JS - 1)
    def _():
        iota = jax.lax.broadcasted_iota(jnp.int32, (BT, NPAD), 1)
        for c in range(TB):
            rows = pl.ds(c * BT, BT)
            p0 = p0_ref[rows, :]
            p1 = p1_ref[rows, :]
            s0 = s0_ref[rows, :]
            s1 = s1_ref[rows, :]
            comb = (jnp.where(iota == p0, s0, 0.0)
                    + jnp.where(iota == p1, s1, 0.0)).astype(_BF16)
            moe = jnp.dot(comb, eo_ref[...], preferred_element_type=_F32)
            o_ref[rows, :] += moe


def _shared_combine(xf_bf, ws_gate, ws_up, ws_down, eo, p0, p1, s0, s1):
    return pl.pallas_call(
        _shared_body,
        grid=(JS,),
        in_specs=[
            pl.BlockSpec((S, D), lambda j: (0, 0)),
            pl.BlockSpec((D, ISC), lambda j: (0, j)),
            pl.BlockSpec((D, ISC), lambda j: (0, j)),
            pl.BlockSpec((ISC, D), lambda j: (j, 0)),
            pl.BlockSpec((NPAD, D), lambda j: (0, 0)),
            pl.BlockSpec((S, 1), lambda j: (0, 0)),
            pl.BlockSpec((S, 1), lambda j: (0, 0)),
            pl.BlockSpec((S, 1), lambda j: (0, 0)),
            pl.BlockSpec((S, 1), lambda j: (0, 0)),
        ],
        out_specs=pl.BlockSpec((S, D), lambda j: (0, 0)),
        out_shape=jax.ShapeDtypeStruct((S, D), _F32),
        compiler_params=pltpu.CompilerParams(
            dimension_semantics=("arbitrary",),
        ),
    )(xf_bf, ws_gate, ws_up, ws_down, eo, p0, p1, s0, s1)


def kernel(hidden_states, w_gate, w1, w2, ws_gate, ws_up, ws_down):
    shape = hidden_states.shape
    xf = hidden_states.reshape(-1, D)

    xf_bf, p0, p1, s0, s1, be = _router(xf, w_gate.T)

    eo = _expert_mlp(be, p0.reshape(1, S), p1.reshape(1, S),
                     xf_bf, w1, w2.astype(_BF16))

    out = _shared_combine(xf_bf, ws_gate, ws_up, ws_down, eo, p0, p1, s0, s1)
    return out.reshape(shape)


# speedup vs baseline: 1.0621x; 1.0621x over previous
import jax
import jax.numpy as jnp
from jax.experimental import pallas as pl
from jax.experimental.pallas import tpu as pltpu

E = 8
TOPK = 2
D = 1024
I = 2048
IS = 4096
S = 2048
BT = 256
NB = S * TOPK // BT + E - 1
NPAD = NB * BT
BTC = 512
TB = S // BTC

_F32 = jnp.float32
_BF16 = jnp.bfloat16


def _router_body(x_ref, wgt_ref, xbf_ref, p0_ref, p1_ref, s0_ref, s1_ref,
                 be_ref):
    x = x_ref[...]
    xbf_ref[...] = x.astype(_BF16)

    logits = jnp.dot(x, wgt_ref[...], preferred_element_type=_F32)
    iota_e = jax.lax.broadcasted_iota(jnp.int32, (S, E), 1)
    v0 = jnp.max(logits, axis=1, keepdims=True)
    i0 = jnp.min(jnp.where(logits == v0, iota_e, E), axis=1, keepdims=True)
    masked = jnp.where(iota_e == i0, -jnp.inf, logits)
    v1 = jnp.max(masked, axis=1, keepdims=True)
    i1 = jnp.min(jnp.where(masked == v1, iota_e, E), axis=1, keepdims=True)
    s0 = 1.0 / (1.0 + jnp.exp(v1 - v0))
    s0_ref[...] = s0
    s1_ref[...] = 1.0 - s0

    oh0 = (iota_e == i0).astype(_BF16)
    oh1 = (iota_e == i1).astype(_BF16)
    ohsum = oh0 + oh1
    iota_r = jax.lax.broadcasted_iota(jnp.int32, (S, S), 0)
    iota_c = jax.lax.broadcasted_iota(jnp.int32, (S, S), 1)
    tri = (iota_r > iota_c).astype(_BF16)
    cum_excl = jnp.dot(tri, ohsum, preferred_element_type=_F32)
    counts = jnp.sum(ohsum.astype(_F32), axis=0, keepdims=True)
    rank0 = jnp.sum(jnp.where(iota_e == i0, cum_excl, 0.0), axis=1,
                    keepdims=True)
    rank1 = jnp.sum(jnp.where(iota_e == i1, cum_excl, 0.0), axis=1,
                    keepdims=True)
    blocks = jnp.floor((counts + (BT - 1)) * (1.0 / BT))
    iota_ec = jax.lax.broadcasted_iota(jnp.int32, (E, E), 0)
    iota_er = jax.lax.broadcasted_iota(jnp.int32, (E, E), 1)
    tlow = (iota_ec <= iota_er).astype(_F32)
    ends_blocks = jnp.dot(blocks, tlow, preferred_element_type=_F32)
    base_rows = (ends_blocks - blocks) * BT
    base_b = jnp.broadcast_to(base_rows, (S, E))
    base0 = jnp.sum(jnp.where(iota_e == i0, base_b, 0.0), axis=1,
                    keepdims=True)
    base1 = jnp.sum(jnp.where(iota_e == i1, base_b, 0.0), axis=1,
                    keepdims=True)
    p0_ref[...] = (base0 + rank0).astype(jnp.int32)
    p1_ref[...] = (base1 + rank1).astype(jnp.int32)

    iota_g = jax.lax.broadcasted_iota(jnp.int32, (128, E), 0)
    ends_b = jnp.broadcast_to(ends_blocks, (128, E))
    be_ref[...] = jnp.sum(
        (iota_g.astype(_F32) >= ends_b).astype(jnp.int32),
        axis=1, keepdims=True)


def _router(xf, w_gate_t):
    return pl.pallas_call(
        _router_body,
        out_shape=(
            jax.ShapeDtypeStruct((S, D), _BF16),
            jax.ShapeDtypeStruct((S, 1), jnp.int32),
            jax.ShapeDtypeStruct((S, 1), jnp.int32),
            jax.ShapeDtypeStruct((S, 1), _F32),
            jax.ShapeDtypeStruct((S, 1), _F32),
            jax.ShapeDtypeStruct((128, 1), jnp.int32),
        ),
    )(xf, w_gate_t)


def _expert_body(be_ref, p0_ref, p1_ref, xf_ref, w1_ref, w2_ref, out_ref):
    g = pl.program_id(0)

    @pl.when(be_ref[g, 0] >= E)
    def _():
        out_ref[...] = jnp.zeros((BT, D), _BF16)

    @pl.when(be_ref[g, 0] < E)
    def _():
        rowpos = g * BT + jax.lax.broadcasted_iota(jnp.int32, (BT, 1), 0)
        p0 = p0_ref[...]
        p1 = p1_ref[...]
        onehot = ((p0 == rowpos).astype(_BF16)
                  + (p1 == rowpos).astype(_BF16))
        xg = jnp.dot(onehot, xf_ref[...], preferred_element_type=_F32)
        w1b = w1_ref[0].astype(_BF16)
        h = jnp.dot(xg.astype(_BF16), w1b, preferred_element_type=_F32)
        a = h[:, :I]
        b = h[:, I:]
        act = (a * jax.nn.sigmoid(a) * b).astype(_BF16)
        out_ref[...] = jnp.dot(act, w2_ref[0],
                               preferred_element_type=_F32).astype(_BF16)


def _expert_mlp(block_expert, p0l, p1l, xf_bf, w1_bf, w2_bf):
    grid_spec = pltpu.PrefetchScalarGridSpec(
        num_scalar_prefetch=1,
        grid=(NB,),
        in_specs=[
            pl.BlockSpec((1, S), lambda g, be: (0, 0)),
            pl.BlockSpec((1, S), lambda g, be: (0, 0)),
            pl.BlockSpec((S, D), lambda g, be: (0, 0)),
            pl.BlockSpec((1, D, 2 * I),
                         lambda g, be: (jnp.minimum(be[g, 0], E - 1), 0, 0)),
            pl.BlockSpec((1, I, D),
                         lambda g, be: (jnp.minimum(be[g, 0], E - 1), 0, 0)),
        ],
        out_specs=pl.BlockSpec((BT, D), lambda g, be: (g, 0)),
    )
    return pl.pallas_call(
        _expert_body,
        grid_spec=grid_spec,
        out_shape=jax.ShapeDtypeStruct((NPAD, D), _BF16),
        compiler_params=pltpu.CompilerParams(
            dimension_semantics=("parallel",),
        ),
    )(block_expert, p0l, p1l, xf_bf, w1_bf, w2_bf)


ISC = 512
JS = IS // ISC


def _shared_body(x_ref, wg_ref, wu_ref, wd_ref, o_ref):
    j = pl.program_id(0)
    x = x_ref[...]
    wgj = wg_ref[...].astype(_BF16)
    wuj = wu_ref[...].astype(_BF16)
    wdj = wd_ref[...].astype(_BF16)
    g_ = jnp.dot(x, wgj, preferred_element_type=_F32)
    u_ = jnp.dot(x, wuj, preferred_element_type=_F32)
    act = (g_ * jax.lax.logistic(g_) * u_).astype(_BF16)
    partial = jnp.dot(act, wdj, preferred_element_type=_F32)

    @pl.when(j == 0)
    def _():
        o_ref[...] = partial

    @pl.when(j > 0)
    def _():
        o_ref[...] += partial


def _shared_mlp(xf_bf, ws_gate, ws_up, ws_down):
    return pl.pallas_call(
        _shared_body,
        grid=(JS,),
        in_specs=[
            pl.BlockSpec((S, D), lambda j: (0, 0)),
            pl.BlockSpec((D, ISC), lambda j: (0, j)),
            pl.BlockSpec((D, ISC), lambda j: (0, j)),
            pl.BlockSpec((ISC, D), lambda j: (j, 0)),
        ],
        out_specs=pl.BlockSpec((S, D), lambda j: (0, 0)),
        out_shape=jax.ShapeDtypeStruct((S, D), _F32),
        compiler_params=pltpu.CompilerParams(
            dimension_semantics=("arbitrary",),
        ),
    )(xf_bf, ws_gate, ws_up, ws_down)


def _combine_body(sh_ref, eo_ref, p0_ref, p1_ref, s0_ref, s1_ref, o_ref):
    p0 = p0_ref[0]
    p1 = p1_ref[0]
    s0 = s0_ref[0]
    s1 = s1_ref[0]
    iota = jax.lax.broadcasted_iota(jnp.int32, (BTC, NPAD), 1)
    comb = (jnp.where(iota == p0, s0, 0.0)
            + jnp.where(iota == p1, s1, 0.0)).astype(_BF16)
    moe = jnp.dot(comb, eo_ref[...], preferred_element_type=_F32)
    o_ref[...] = sh_ref[...] + moe


def _combine(shared, eo, p0, p1, s0, s1):
    return pl.pallas_call(
        _combine_body,
        grid=(TB,),
        in_specs=[
            pl.BlockSpec((BTC, D), lambda t: (t, 0)),
            pl.BlockSpec((NPAD, D), lambda t: (0, 0)),
            pl.BlockSpec((1, BTC, 1), lambda t: (t, 0, 0)),
            pl.BlockSpec((1, BTC, 1), lambda t: (t, 0, 0)),
            pl.BlockSpec((1, BTC, 1), lambda t: (t, 0, 0)),
            pl.BlockSpec((1, BTC, 1), lambda t: (t, 0, 0)),
        ],
        out_specs=pl.BlockSpec((BTC, D), lambda t: (t, 0)),
        out_shape=jax.ShapeDtypeStruct((S, D), _F32),
        compiler_params=pltpu.CompilerParams(
            dimension_semantics=("arbitrary",),
        ),
    )(shared, eo, p0, p1, s0, s1)


def kernel(hidden_states, w_gate, w1, w2, ws_gate, ws_up, ws_down):
    shape = hidden_states.shape
    xf = hidden_states.reshape(-1, D)

    xf_bf, p0, p1, s0, s1, be = _router(xf, w_gate.T)

    eo = _expert_mlp(be, p0.reshape(1, S), p1.reshape(1, S),
                     xf_bf, w1, w2.astype(_BF16))

    shared = _shared_mlp(xf_bf, ws_gate, ws_up, ws_down)

    out = _combine(
        shared, eo,
        p0.reshape(TB, BTC, 1), p1.reshape(TB, BTC, 1),
        s0.reshape(TB, BTC, 1), s1.reshape(TB, BTC, 1))
    return out.reshape(shape)
